# baseline (speedup 1.0000x reference)
import jax
import jax.numpy as jnp
from jax import lax
from jax.experimental import pallas as pl
from jax.experimental.pallas import tpu as pltpu

N = 10000
D = 256
G = 128
GU = 128
HID = 512
OUT = 256
BN = 2000
NBLK = N // BN

_DN_T = (((1,), (1,)), ((), ()))


def _fused_kernel(batch_ref, x_ref, u_hbm, w1_hbm, b1_hbm, w2_hbm, b2_hbm,
                  out_ref, acc_ref, cnt_ref, u_v, w1_v, b1_v, w2_v, b2_v,
                  sem):
    i = pl.program_id(0)

    @pl.when(i == 0)
    def _init():
        acc_ref[...] = jnp.zeros_like(acc_ref)
        cnt_ref[...] = jnp.zeros_like(cnt_ref)
        pltpu.make_async_copy(u_hbm, u_v, sem).start()
        pltpu.make_async_copy(w1_hbm, w1_v, sem).start()
        pltpu.make_async_copy(b1_hbm, b1_v, sem).start()
        pltpu.make_async_copy(w2_hbm, w2_v, sem).start()
        pltpu.make_async_copy(b2_hbm, b2_v, sem).start()

    seg = batch_ref[0]
    seg_b = jnp.broadcast_to(seg, (G, BN))
    gids = lax.broadcasted_iota(jnp.int32, (G, BN), 0)
    onehot_t = (gids == seg_b).astype(jnp.bfloat16)

    acc_ref[...] += jnp.dot(onehot_t, x_ref[...].astype(jnp.bfloat16),
                            preferred_element_type=jnp.float32)
    cnt_ref[...] += jnp.sum(onehot_t.astype(jnp.float32), axis=1,
                            keepdims=True)

    @pl.when(i == NBLK - 1)
    def _finish():
        pltpu.make_async_copy(u_hbm, u_v, sem).wait()
        pltpu.make_async_copy(w1_hbm, w1_v, sem).wait()
        pltpu.make_async_copy(b1_hbm, b1_v, sem).wait()
        pltpu.make_async_copy(w2_hbm, w2_v, sem).wait()
        pltpu.make_async_copy(b2_hbm, b2_v, sem).wait()
        mean = acc_ref[...] / jnp.clip(cnt_ref[...], 1.0, None)
        h = lax.dot_general(u_v[...], w1_v[:, :GU], _DN_T,
                            preferred_element_type=jnp.float32)
        h += lax.dot_general(mean, w1_v[:, GU:], _DN_T,
                             preferred_element_type=jnp.float32)
        h = jnp.maximum(h + b1_v[...], 0.0)
        y = lax.dot_general(h, w2_v[...], _DN_T,
                            preferred_element_type=jnp.float32)
        out_ref[...] = y + b2_v[...]


def kernel(x, edge_index, edge_attr, u, batch, W1, b1, W2, b2):
    del edge_index, edge_attr
    batch3 = batch.reshape(NBLK, 1, BN)
    b1r = b1.reshape(1, HID)
    b2r = b2.reshape(1, OUT)

    any_spec = pl.BlockSpec(memory_space=pl.ANY)
    return pl.pallas_call(
        _fused_kernel,
        grid=(NBLK,),
        in_specs=[
            pl.BlockSpec((1, 1, BN), lambda i: (i, 0, 0)),
            pl.BlockSpec((BN, D), lambda i: (i, 0)),
            any_spec, any_spec, any_spec, any_spec, any_spec,
        ],
        out_specs=pl.BlockSpec((G, OUT), lambda i: (0, 0)),
        out_shape=jax.ShapeDtypeStruct((G, OUT), jnp.float32),
        scratch_shapes=[
            pltpu.VMEM((G, D), jnp.float32),
            pltpu.VMEM((G, 1), jnp.float32),
            pltpu.VMEM((G, GU), jnp.float32),
            pltpu.VMEM((HID, GU + D), jnp.float32),
            pltpu.VMEM((1, HID), jnp.float32),
            pltpu.VMEM((OUT, HID), jnp.float32),
            pltpu.VMEM((1, OUT), jnp.float32),
            pltpu.SemaphoreType.DMA,
        ],
        compiler_params=pltpu.CompilerParams(
            dimension_semantics=("arbitrary",),
        ),
    )(batch3, x, u, W1, b1r, W2, b2r)


# speedup vs baseline: 1.1845x; 1.1845x over previous
import jax
import jax.numpy as jnp
from jax import lax
from jax.experimental import pallas as pl
from jax.experimental.pallas import tpu as pltpu

N = 10000
D = 256
G = 128
GU = 128
HID = 512
OUT = 256
BN = 5000
NBLK = N // BN

_DN_T = (((1,), (1,)), ((), ()))


def _fused_kernel(batch_ref, x_ref, u_hbm, w1_hbm, b1_hbm, w2_hbm, b2_hbm,
                  out_ref, acc_ref, cnt_ref, u_v, w1_v, b1_v, w2_v, b2_v,
                  sem):
    i = pl.program_id(0)

    @pl.when(i == 0)
    def _init():
        acc_ref[...] = jnp.zeros_like(acc_ref)
        cnt_ref[...] = jnp.zeros_like(cnt_ref)
        pltpu.make_async_copy(u_hbm, u_v, sem).start()
        pltpu.make_async_copy(w1_hbm, w1_v, sem).start()
        pltpu.make_async_copy(b1_hbm, b1_v, sem).start()
        pltpu.make_async_copy(w2_hbm, w2_v, sem).start()
        pltpu.make_async_copy(b2_hbm, b2_v, sem).start()

    seg = batch_ref[0]
    seg_b = jnp.broadcast_to(seg, (G, BN))
    gids = lax.broadcasted_iota(jnp.int32, (G, BN), 0)
    onehot_t = (gids == seg_b).astype(jnp.bfloat16)

    acc_ref[...] += jnp.dot(onehot_t, x_ref[...].astype(jnp.bfloat16),
                            preferred_element_type=jnp.float32)
    cnt_ref[...] += jnp.sum(onehot_t.astype(jnp.float32), axis=1,
                            keepdims=True)

    @pl.when(i == NBLK - 1)
    def _finish():
        pltpu.make_async_copy(u_hbm, u_v, sem).wait()
        pltpu.make_async_copy(w1_hbm, w1_v, sem).wait()
        pltpu.make_async_copy(b1_hbm, b1_v, sem).wait()
        pltpu.make_async_copy(w2_hbm, w2_v, sem).wait()
        pltpu.make_async_copy(b2_hbm, b2_v, sem).wait()
        mean = acc_ref[...] / jnp.clip(cnt_ref[...], 1.0, None)
        h = lax.dot_general(u_v[...], w1_v[:, :GU], _DN_T,
                            preferred_element_type=jnp.float32)
        h += lax.dot_general(mean, w1_v[:, GU:], _DN_T,
                             preferred_element_type=jnp.float32)
        h = jnp.maximum(h + b1_v[...], 0.0)
        y = lax.dot_general(h, w2_v[...], _DN_T,
                            preferred_element_type=jnp.float32)
        out_ref[...] = y + b2_v[...]


def kernel(x, edge_index, edge_attr, u, batch, W1, b1, W2, b2):
    del edge_index, edge_attr
    batch3 = batch.reshape(NBLK, 1, BN)
    b1r = b1.reshape(1, HID)
    b2r = b2.reshape(1, OUT)

    any_spec = pl.BlockSpec(memory_space=pl.ANY)
    return pl.pallas_call(
        _fused_kernel,
        grid=(NBLK,),
        in_specs=[
            pl.BlockSpec((1, 1, BN), lambda i: (i, 0, 0)),
            pl.BlockSpec((BN, D), lambda i: (i, 0)),
            any_spec, any_spec, any_spec, any_spec, any_spec,
        ],
        out_specs=pl.BlockSpec((G, OUT), lambda i: (0, 0)),
        out_shape=jax.ShapeDtypeStruct((G, OUT), jnp.float32),
        scratch_shapes=[
            pltpu.VMEM((G, D), jnp.float32),
            pltpu.VMEM((G, 1), jnp.float32),
            pltpu.VMEM((G, GU), jnp.float32),
            pltpu.VMEM((HID, GU + D), jnp.float32),
            pltpu.VMEM((1, HID), jnp.float32),
            pltpu.VMEM((OUT, HID), jnp.float32),
            pltpu.VMEM((1, OUT), jnp.float32),
            pltpu.SemaphoreType.DMA,
        ],
        compiler_params=pltpu.CompilerParams(
            dimension_semantics=("arbitrary",),
        ),
    )(batch3, x, u, W1, b1r, W2, b2r)
